# baseline (speedup 1.0000x reference)
import jax
import jax.numpy as jnp
from jax import lax
from jax.experimental import pallas as pl
from jax.experimental.pallas import tpu as pltpu, tpu_sc as plsc

ITEM_NUM = 1000000
ATTR_NUM = 100000
RATING_NUM = 5
EMBED_DIM = 32
ATTR_FNUM = 2
MAX_HIST_LEN = 200
BATCH = 4096
FIELD_NUM = 7

NC = 2
NS = 16
NW = NC * NS
B_PER_W = BATCH // NW
L = MAX_HIST_LEN
GL = 25
NG = L // GL
INV_L = 1.0 / MAX_HIST_LEN


def _zeros():
    return jnp.zeros((16,), jnp.float32)


def _sc_body(hiT, haP, hrT, iid_hbm, aidT,
             item_t, attr_t, rating_t, out_hbm,
             outbuf, rt_v, ii_v, av0, av1,
             itg0, itg1, a0g0, a0g1, a1g0, a1g1, rtg0, rtg1,
             irows0, irows1, a0rows0, a0rows1, a1rows0, a1rows1,
             acc_i, acc_a0, acc_a1, counts,
             sem_g0, sem_g1, sem_r0, sem_r1, sem_a):
    itg = (itg0, itg1)
    a0g = (a0g0, a0g1)
    a1g = (a1g0, a1g1)
    rtg = (rtg0, rtg1)
    irows = (irows0, irows1)
    a0rows = (a0rows0, a0rows1)
    a1rows = (a1rows0, a1rows1)
    sem_g = (sem_g0, sem_g1)
    sem_r = (sem_r0, sem_r1)

    wid = lax.axis_index("s") * NC + lax.axis_index("c")
    base = wid * B_PER_W

    def start_group(g, gsl):
        l0 = g * GL
        pltpu.async_copy(hiT.at[pl.ds(l0, GL), pl.ds(base, B_PER_W)],
                         itg[gsl], sem_g[gsl])
        pltpu.async_copy(haP.at[pl.ds(l0, GL), 0, pl.ds(base, B_PER_W)],
                         a0g[gsl], sem_g[gsl])
        pltpu.async_copy(haP.at[pl.ds(l0, GL), 1, pl.ds(base, B_PER_W)],
                         a1g[gsl], sem_g[gsl])
        pltpu.async_copy(hrT.at[pl.ds(l0, GL), pl.ds(base, B_PER_W)],
                         rtg[gsl], sem_g[gsl])

    def wait_group(gsl):
        for dst in (itg[gsl], a0g[gsl], a1g[gsl], rtg[gsl]):
            pltpu.make_async_copy(
                hiT.at[pl.ds(0, GL), pl.ds(0, B_PER_W)], dst,
                sem_g[gsl]).wait()

    def start_gathers(gsl, row, slot):
        pltpu.async_copy(item_t.at[itg[gsl].at[row]], irows[slot],
                         sem_r[slot])
        pltpu.async_copy(attr_t.at[a0g[gsl].at[row]], a0rows[slot],
                         sem_r[slot])
        pltpu.async_copy(attr_t.at[a1g[gsl].at[row]], a1rows[slot],
                         sem_r[slot])

    def wait_gathers(slot):
        for dst in (irows[slot], a0rows[slot], a1rows[slot]):
            pltpu.make_async_copy(item_t.at[pl.ds(0, B_PER_W)], dst,
                                  sem_r[slot]).wait()

    pltpu.sync_copy(rating_t, rt_v)
    start_group(0, 0)
    start_group(1, 1)

    pltpu.sync_copy(iid_hbm.at[pl.ds(base, B_PER_W)], ii_v)
    pltpu.sync_copy(aidT.at[0, pl.ds(base, B_PER_W)], av0)
    pltpu.sync_copy(aidT.at[1, pl.ds(base, B_PER_W)], av1)
    pltpu.async_copy(item_t.at[ii_v], irows0, sem_a)
    pltpu.async_copy(attr_t.at[av0], a0rows0, sem_a)
    pltpu.async_copy(attr_t.at[av1], a1rows0, sem_a)
    for dst in (irows0, a0rows0, a1rows0):
        pltpu.make_async_copy(item_t.at[pl.ds(0, B_PER_W)], dst,
                              sem_a).wait()

    @pl.loop(0, B_PER_W)
    def _copy_single(i):
        for v in range(2):
            sl = pl.ds(v * 16, 16)
            outbuf[i, 0, sl] = irows0[i, sl]
            outbuf[i, 1, sl] = a0rows0[i, sl]
            outbuf[i, 2, sl] = a1rows0[i, sl]

    znil = _zeros()
    inil = jnp.zeros((16,), jnp.int32)

    @pl.loop(0, B_PER_W)
    def _zero_acc(b):
        for v in range(2):
            sl = pl.ds(v * 16, 16)
            acc_i[b, sl] = znil
            acc_a0[b, sl] = znil
            acc_a1[b, sl] = znil

    for q in range(5 * 8):
        counts[q, :] = inil

    one = jnp.ones((16,), jnp.int32)

    def accumulate(gsl, row, slot):
        @pl.loop(0, B_PER_W, unroll=4)
        def _acc(b):
            for v in range(2):
                sl = pl.ds(v * 16, 16)
                plsc.addupdate(acc_i.at[b, sl], irows[slot][b, sl])
                plsc.addupdate(acc_a0.at[b, sl], a0rows[slot][b, sl])
                plsc.addupdate(acc_a1.at[b, sl], a1rows[slot][b, sl])
        for vv in range(8):
            rv = rtg[gsl][row, pl.ds(vv * 16, 16)]
            for r in range(RATING_NUM):
                plsc.addupdate(counts.at[r * 8 + vv],
                               jnp.where(rv == r, one, inil))

    wait_group(0)
    start_gathers(0, 0, 0)

    for g in range(NG):
        gsl = g % 2

        @pl.loop(0, GL - 1, step=2)
        def _inner(j):
            for t in range(2):
                slot = (g + t) % 2
                wait_gathers(slot)
                start_gathers(gsl, j + t + 1, 1 - slot)
                accumulate(gsl, j + t, slot)

        slot = (g + GL - 1) % 2
        wait_gathers(slot)
        if g + 2 < NG:
            start_group(g + 2, gsl)
        if g + 1 < NG:
            wait_group(1 - gsl)
            start_gathers(1 - gsl, 0, 1 - slot)
        accumulate(gsl, GL - 1, slot)

    @pl.loop(0, B_PER_W)
    def _finish(b):
        s0, s1 = pl.ds(0, 16), pl.ds(16, 16)
        outbuf[b, 3, s0] = acc_i[b, s0] * INV_L
        outbuf[b, 3, s1] = acc_i[b, s1] * INV_L
        outbuf[b, 4, s0] = acc_a0[b, s0] * INV_L
        outbuf[b, 4, s1] = acc_a0[b, s1] * INV_L
        outbuf[b, 5, s0] = acc_a1[b, s0] * INV_L
        outbuf[b, 5, s1] = acc_a1[b, s1] * INV_L
        off = jnp.zeros((16,), jnp.int32) + (b % 16)
        acc6 = [_zeros(), _zeros()]
        for r in range(RATING_NUM):
            cvec = counts[r * 8 + b // 16]
            w = jnp.take_along_axis(cvec, off, axis=0).astype(jnp.float32)
            w = w * INV_L
            for v in range(2):
                acc6[v] += w * rt_v[r, pl.ds(v * 16, 16)]
        outbuf[b, 6, s0] = acc6[0]
        outbuf[b, 6, s1] = acc6[1]

    pltpu.sync_copy(outbuf, out_hbm.at[pl.ds(base, B_PER_W)])


@jax.jit
def _run(hiT, haP, hrT, iid_a, aidT, item_table, attr_table, rating_table):
    mesh = plsc.VectorSubcoreMesh(core_axis_name="c", subcore_axis_name="s")
    f = pl.kernel(
        _sc_body,
        out_type=jax.ShapeDtypeStruct((BATCH, FIELD_NUM, EMBED_DIM),
                                      jnp.float32),
        mesh=mesh,
        scratch_types=[
            pltpu.VMEM((B_PER_W, FIELD_NUM, EMBED_DIM), jnp.float32),
            pltpu.VMEM((RATING_NUM + 1, EMBED_DIM), jnp.float32),
            pltpu.VMEM((B_PER_W,), jnp.int32),
            pltpu.VMEM((B_PER_W,), jnp.int32),
            pltpu.VMEM((B_PER_W,), jnp.int32),
            pltpu.VMEM((GL, B_PER_W), jnp.int32),
            pltpu.VMEM((GL, B_PER_W), jnp.int32),
            pltpu.VMEM((GL, B_PER_W), jnp.int32),
            pltpu.VMEM((GL, B_PER_W), jnp.int32),
            pltpu.VMEM((GL, B_PER_W), jnp.int32),
            pltpu.VMEM((GL, B_PER_W), jnp.int32),
            pltpu.VMEM((GL, B_PER_W), jnp.int32),
            pltpu.VMEM((GL, B_PER_W), jnp.int32),
            pltpu.VMEM((B_PER_W, EMBED_DIM), jnp.float32),
            pltpu.VMEM((B_PER_W, EMBED_DIM), jnp.float32),
            pltpu.VMEM((B_PER_W, EMBED_DIM), jnp.float32),
            pltpu.VMEM((B_PER_W, EMBED_DIM), jnp.float32),
            pltpu.VMEM((B_PER_W, EMBED_DIM), jnp.float32),
            pltpu.VMEM((B_PER_W, EMBED_DIM), jnp.float32),
            pltpu.VMEM((B_PER_W, EMBED_DIM), jnp.float32),
            pltpu.VMEM((B_PER_W, EMBED_DIM), jnp.float32),
            pltpu.VMEM((B_PER_W, EMBED_DIM), jnp.float32),
            pltpu.VMEM((5 * 8, 16), jnp.int32),
            pltpu.SemaphoreType.DMA,
            pltpu.SemaphoreType.DMA,
            pltpu.SemaphoreType.DMA,
            pltpu.SemaphoreType.DMA,
            pltpu.SemaphoreType.DMA,
        ],
        compiler_params=pltpu.CompilerParams(use_tc_tiling_on_sc=False),
    )
    return f(hiT, haP, hrT, iid_a, aidT, item_table, attr_table,
             rating_table)


def kernel(hist_iid_seq, hist_aid_seq, hist_rate_seq, hist_seq_len, iid, aid,
           lb, item_table, attr_table, rating_table):
    del hist_seq_len, lb
    hiT = jnp.transpose(hist_iid_seq.astype(jnp.int32), (1, 0))
    haP = jnp.transpose(hist_aid_seq.astype(jnp.int32), (1, 2, 0))
    hrT = jnp.transpose(hist_rate_seq.astype(jnp.int32), (1, 0))
    aidT = jnp.transpose(aid.astype(jnp.int32), (1, 0))
    return _run(hiT, haP, hrT, iid.astype(jnp.int32), aidT,
                item_table.astype(jnp.float32),
                attr_table.astype(jnp.float32),
                rating_table.astype(jnp.float32))


# speedup vs baseline: 1.1252x; 1.1252x over previous
import jax
import jax.numpy as jnp
from jax import lax
from jax.experimental import pallas as pl
from jax.experimental.pallas import tpu as pltpu, tpu_sc as plsc

ITEM_NUM = 1000000
ATTR_NUM = 100000
RATING_NUM = 5
EMBED_DIM = 32
ATTR_FNUM = 2
MAX_HIST_LEN = 200
BATCH = 4096
FIELD_NUM = 7

NC = 2
NS = 16
NW = NC * NS
B_PER_W = BATCH // NW
L = MAX_HIST_LEN
LP = 256
APL = 512
INV_L = 1.0 / MAX_HIST_LEN

ITEM_CHUNKS = ((0, 104), (104, 96))
ATTR_CHUNKS = ((0, 104), (104, 104), (208, 104), (312, 88))


def _zeros():
    return jnp.zeros((16,), jnp.float32)


def _worker_base():
    wid = lax.axis_index("s") * NC + lax.axis_index("c")
    return wid, wid * B_PER_W


def _attr_body(ha_hbm, hr_hbm, aid_hbm, attr_t, rating_t, out_hbm,
               outbuf, rt_v, av_v,
               ai0, ai1, ri0, ri1, arow0, arow1,
               sem_idx0, sem_idx1, sem_rows0, sem_rows1, sem_a):
    attr_idx = (ai0, ai1)
    rate_idx = (ri0, ri1)
    attr_rows = (arow0, arow1)
    sem_idx = (sem_idx0, sem_idx1)
    sem_rows = (sem_rows0, sem_rows1)

    wid, base = _worker_base()

    pltpu.sync_copy(rating_t, rt_v)

    pltpu.sync_copy(aid_hbm.at[wid], av_v)
    for c in range(2):
        pltpu.async_copy(attr_t.at[av_v.at[c]],
                         arow0.at[pl.ds(c * 128, 128)], sem_a)
    for c in range(2):
        pltpu.make_async_copy(attr_t.at[pl.ds(0, 128)],
                              arow0.at[pl.ds(c * 128, 128)], sem_a).wait()

    @pl.loop(0, B_PER_W)
    def _copy_single(i):
        for v in range(2):
            sl = pl.ds(v * 16, 16)
            outbuf[i, 0, sl] = arow0[2 * i, sl]
            outbuf[i, 1, sl] = arow0[2 * i + 1, sl]

    def start_idx(gb, slot):
        pltpu.async_copy(ha_hbm.at[gb], attr_idx[slot], sem_idx[slot])
        pltpu.async_copy(hr_hbm.at[gb], rate_idx[slot], sem_idx[slot])

    def wait_idx(slot):
        pltpu.make_async_copy(ha_hbm.at[0], attr_idx[slot],
                              sem_idx[slot]).wait()
        pltpu.make_async_copy(hr_hbm.at[0], rate_idx[slot],
                              sem_idx[slot]).wait()

    def start_gathers(slot):
        for off, ln in ATTR_CHUNKS:
            pltpu.async_copy(attr_t.at[attr_idx[slot].at[pl.ds(off, ln)]],
                             attr_rows[slot].at[pl.ds(off, ln)],
                             sem_rows[slot])

    def wait_gathers(slot):
        for off, ln in ATTR_CHUNKS:
            pltpu.make_async_copy(attr_t.at[pl.ds(0, ln)],
                                  attr_rows[slot].at[pl.ds(off, ln)],
                                  sem_rows[slot]).wait()

    def rating(k, slot):
        counts = [jnp.zeros((16,), jnp.int32) for _ in range(RATING_NUM)]
        one = jnp.ones((16,), jnp.int32)
        nil = jnp.zeros((16,), jnp.int32)
        lane = lax.broadcasted_iota(jnp.int32, (16,), 0)
        for i in range(13):
            rv = rate_idx[slot][pl.ds(i * 16, 16)]
            for r in range(RATING_NUM):
                counts[r] = counts[r] + jnp.where(rv == r, one, nil)
        acc = [_zeros(), _zeros()]
        for r in range(RATING_NUM):
            tot = counts[r]
            for sh in (8, 4, 2, 1):
                tot = tot + jnp.take_along_axis(tot, lane ^ sh, axis=0)
            w = tot.astype(jnp.float32) * INV_L
            for v in range(2):
                acc[v] += w * rt_v[r, pl.ds(v * 16, 16)]
        for v in range(2):
            outbuf[k, 4, pl.ds(v * 16, 16)] = acc[v]

    def reduce(k, slot):
        ar = attr_rows[slot]

        def body(l, accs):
            a00, a01, a10, a11 = accs
            s0, s1 = pl.ds(0, 16), pl.ds(16, 16)
            a00 = a00 + ar[2 * l, s0]
            a01 = a01 + ar[2 * l, s1]
            a10 = a10 + ar[2 * l + 1, s0]
            a11 = a11 + ar[2 * l + 1, s1]
            return a00, a01, a10, a11

        init = (_zeros(), _zeros(), _zeros(), _zeros())
        a00, a01, a10, a11 = lax.fori_loop(0, L, body, init, unroll=4)
        s0, s1 = pl.ds(0, 16), pl.ds(16, 16)
        outbuf[k, 2, s0] = a00 * INV_L
        outbuf[k, 2, s1] = a01 * INV_L
        outbuf[k, 3, s0] = a10 * INV_L
        outbuf[k, 3, s1] = a11 * INV_L

    def step(k, slot, do_idx, do_gather):
        wait_gathers(slot)
        rating(k, slot)
        if do_idx:
            start_idx(base + k + 2, slot)
        if do_gather:
            wait_idx(1 - slot)
            start_gathers(1 - slot)
        reduce(k, slot)

    start_idx(base + 0, 0)
    start_idx(base + 1, 1)
    wait_idx(0)
    start_gathers(0)

    @pl.loop(0, B_PER_W - 4, step=2)
    def _main(k):
        step(k, 0, True, True)
        step(k + 1, 1, True, True)

    step(B_PER_W - 4, 0, True, True)
    step(B_PER_W - 3, 1, True, True)
    step(B_PER_W - 2, 0, False, True)
    step(B_PER_W - 1, 1, False, False)

    pltpu.sync_copy(outbuf, out_hbm.at[pl.ds(base, B_PER_W)])


def _item_body(hi_hbm, iid_hbm, item_t, out_hbm,
               outbuf, ii_v,
               ii0, ii1, irow0, irow1,
               sem_idx0, sem_idx1, sem_rows0, sem_rows1, sem_a):
    item_idx = (ii0, ii1)
    item_rows = (irow0, irow1)
    sem_idx = (sem_idx0, sem_idx1)
    sem_rows = (sem_rows0, sem_rows1)

    _, base = _worker_base()

    pltpu.sync_copy(iid_hbm.at[pl.ds(base, B_PER_W)], ii_v)
    pltpu.async_copy(item_t.at[ii_v], irow0.at[pl.ds(0, 128)], sem_a)
    pltpu.make_async_copy(item_t.at[pl.ds(0, 128)],
                          irow0.at[pl.ds(0, 128)], sem_a).wait()

    @pl.loop(0, B_PER_W)
    def _copy_single(i):
        for v in range(2):
            sl = pl.ds(v * 16, 16)
            outbuf[i, 0, sl] = irow0[i, sl]

    def start_idx(gb, slot):
        pltpu.async_copy(hi_hbm.at[gb], item_idx[slot], sem_idx[slot])

    def wait_idx(slot):
        pltpu.make_async_copy(hi_hbm.at[0], item_idx[slot],
                              sem_idx[slot]).wait()

    def start_gathers(slot):
        for off, ln in ITEM_CHUNKS:
            pltpu.async_copy(item_t.at[item_idx[slot].at[pl.ds(off, ln)]],
                             item_rows[slot].at[pl.ds(off, ln)],
                             sem_rows[slot])

    def wait_gathers(slot):
        for off, ln in ITEM_CHUNKS:
            pltpu.make_async_copy(item_t.at[pl.ds(0, ln)],
                                  item_rows[slot].at[pl.ds(off, ln)],
                                  sem_rows[slot]).wait()

    def reduce(k, slot):
        ir = item_rows[slot]

        def body(l, accs):
            i0, i1 = accs
            s0, s1 = pl.ds(0, 16), pl.ds(16, 16)
            return i0 + ir[l, s0], i1 + ir[l, s1]

        i0, i1 = lax.fori_loop(0, L, body, (_zeros(), _zeros()), unroll=4)
        s0, s1 = pl.ds(0, 16), pl.ds(16, 16)
        outbuf[k, 1, s0] = i0 * INV_L
        outbuf[k, 1, s1] = i1 * INV_L

    def step(k, slot, do_idx, do_gather):
        wait_gathers(slot)
        if do_idx:
            start_idx(base + k + 2, slot)
        if do_gather:
            wait_idx(1 - slot)
            start_gathers(1 - slot)
        reduce(k, slot)

    start_idx(base + 0, 0)
    start_idx(base + 1, 1)
    wait_idx(0)
    start_gathers(0)

    @pl.loop(0, B_PER_W - 4, step=2)
    def _main(k):
        step(k, 0, True, True)
        step(k + 1, 1, True, True)

    step(B_PER_W - 4, 0, True, True)
    step(B_PER_W - 3, 1, True, True)
    step(B_PER_W - 2, 0, False, True)
    step(B_PER_W - 1, 1, False, False)

    pltpu.sync_copy(outbuf, out_hbm.at[pl.ds(base, B_PER_W)])


@jax.jit
def _run(hi_p, ha_p, hr_p, iid_a, aid3, item_table, attr_table,
         rating_table):
    mesh = plsc.VectorSubcoreMesh(core_axis_name="c", subcore_axis_name="s")
    params = pltpu.CompilerParams(use_tc_tiling_on_sc=False)
    attr_f = pl.kernel(
        _attr_body,
        out_type=jax.ShapeDtypeStruct((BATCH, 5, EMBED_DIM), jnp.float32),
        mesh=mesh,
        scratch_types=[
            pltpu.VMEM((B_PER_W, 5, EMBED_DIM), jnp.float32),
            pltpu.VMEM((RATING_NUM + 1, EMBED_DIM), jnp.float32),
            pltpu.VMEM((2, 128), jnp.int32),
            pltpu.VMEM((APL,), jnp.int32),
            pltpu.VMEM((APL,), jnp.int32),
            pltpu.VMEM((LP,), jnp.int32),
            pltpu.VMEM((LP,), jnp.int32),
            pltpu.VMEM((APL, EMBED_DIM), jnp.float32),
            pltpu.VMEM((APL, EMBED_DIM), jnp.float32),
            pltpu.SemaphoreType.DMA,
            pltpu.SemaphoreType.DMA,
            pltpu.SemaphoreType.DMA,
            pltpu.SemaphoreType.DMA,
            pltpu.SemaphoreType.DMA,
        ],
        compiler_params=params,
    )
    item_f = pl.kernel(
        _item_body,
        out_type=jax.ShapeDtypeStruct((BATCH, 2, EMBED_DIM), jnp.float32),
        mesh=mesh,
        scratch_types=[
            pltpu.VMEM((B_PER_W, 2, EMBED_DIM), jnp.float32),
            pltpu.VMEM((B_PER_W,), jnp.int32),
            pltpu.VMEM((LP,), jnp.int32),
            pltpu.VMEM((LP,), jnp.int32),
            pltpu.VMEM((LP, EMBED_DIM), jnp.float32),
            pltpu.VMEM((LP, EMBED_DIM), jnp.float32),
            pltpu.SemaphoreType.DMA,
            pltpu.SemaphoreType.DMA,
            pltpu.SemaphoreType.DMA,
            pltpu.SemaphoreType.DMA,
            pltpu.SemaphoreType.DMA,
        ],
        compiler_params=params,
    )
    out_a = attr_f(ha_p, hr_p, aid3, attr_table, rating_table)
    out_i = item_f(hi_p, iid_a, item_table)
    return jnp.concatenate(
        [out_i[:, 0:1], out_a[:, 0:2], out_i[:, 1:2], out_a[:, 2:5]], axis=1)


def kernel(hist_iid_seq, hist_aid_seq, hist_rate_seq, hist_seq_len, iid, aid,
           lb, item_table, attr_table, rating_table):
    del hist_seq_len, lb
    hi_p = jnp.pad(hist_iid_seq.astype(jnp.int32), ((0, 0), (0, LP - L)))
    ha = hist_aid_seq.astype(jnp.int32).reshape(BATCH, 2 * L)
    ha_p = jnp.pad(ha, ((0, 0), (0, APL - 2 * L)))
    hr_p = jnp.pad(hist_rate_seq.astype(jnp.int32), ((0, 0), (0, LP - L)),
                   constant_values=RATING_NUM)
    aid3 = aid.astype(jnp.int32).reshape(NW, 2, B_PER_W)
    return _run(hi_p, ha_p, hr_p, iid.astype(jnp.int32), aid3,
                item_table.astype(jnp.float32),
                attr_table.astype(jnp.float32),
                rating_table.astype(jnp.float32))
